# baseline (speedup 1.0000x reference)
import functools

import jax
import jax.numpy as jnp
from jax import lax
from jax.experimental import pallas as pl
from jax.experimental.pallas import tpu as pltpu
from jax.experimental.pallas import tpu_sc as plsc

N = 10000
E = 320000
IN_DIM = 128
HID = 64
OUT = 32
NUM_MASK = 5000
NUM_NOISE = 250

NP2 = 10240
ENC_ROW = N
DUMP_ROW = NP2 - 1
NC, NS = 2, 16
NW = NC * NS
CH = 128
SB = 4
EPW = 10240
EP = EPW * NW
NCHUNK = EPW // CH
NSUP = NCHUNK // SB
SUPE = SB * CH
NSUPT = EP // SUPE
SLOW_CID = 1
NSUP_SLOW = 10
NSUP_FAST = 30
RPT = NP2 // NS
RPW = NP2 // NW
BCH = 64
D2 = 2 * HID

_SC_PARAMS = dict(
    compiler_params=pltpu.CompilerParams(needs_layout_passes=False),
)


def _mask_constants():
    rk = jax.random.key(42)
    r1, r2, r3, r4, r5 = jax.random.split(rk, 5)
    perm = jax.random.permutation(r1, N)
    mask_nodes = perm[:NUM_MASK]
    keep_nodes = perm[NUM_MASK:]
    perm_mask = jax.random.permutation(r2, NUM_MASK)
    token_nodes = mask_nodes[perm_mask[: NUM_MASK - NUM_NOISE]]
    noise_nodes = mask_nodes[perm_mask[NUM_MASK - NUM_NOISE:]]
    noise_chosen = jax.random.permutation(r3, N)[:NUM_NOISE]

    remap = jnp.arange(NP2, dtype=jnp.int32)
    remap = remap.at[token_nodes].set(ENC_ROW)
    remap = remap.at[noise_nodes].set(noise_chosen)

    m4 = jnp.zeros((NP2, 1), jnp.float32).at[
        jax.random.permutation(r4, N)[:NUM_MASK]].set(1.0)
    m5 = jnp.zeros((NP2, 1), jnp.float32).at[
        jax.random.permutation(r5, N)[:NUM_MASK]].set(1.0)
    return mask_nodes, keep_nodes, remap, m4, m5


def _sigmoid(x):
    return 1.0 / (1.0 + jnp.exp(-x))


def _bc_i32(x):
    return jax.lax.bitcast_convert_type(x, jnp.int32)


def _elu(x):
    return jnp.where(x > 0, x, jnp.exp(jnp.minimum(x, 0.0)) - 1.0)


def _k1_body(x_ref, w1l_ref, w1r_ref, w1le_ref, w1re_ref, as_ref, ad_ref,
             aes_ref, aed_ref, enc_ref,
             t_ref, b1_ref, c1_ref, be_ref, ce_ref, encr_ref):
    x = x_ref[...]
    f1 = jnp.dot(x, w1l_ref[...], preferred_element_type=jnp.float32)
    fd = jnp.dot(x, w1r_ref[...], preferred_element_type=jnp.float32)
    xe = jnp.dot(x, w1le_ref[...], preferred_element_type=jnp.float32)
    xde = jnp.dot(x, w1re_ref[...], preferred_element_type=jnp.float32)
    t_ref[...] = jnp.concatenate([f1, xe], axis=1)
    b1_ref[...] = jnp.sum(f1 * as_ref[...], axis=1, keepdims=True)
    c1_ref[...] = jnp.sum(fd * ad_ref[...], axis=1, keepdims=True)
    be_ref[...] = jnp.sum(xe * aes_ref[...], axis=1, keepdims=True)
    ce_ref[...] = jnp.sum(xde * aed_ref[...], axis=1, keepdims=True)
    enc = enc_ref[...]
    e1 = jnp.dot(enc, w1l_ref[...], preferred_element_type=jnp.float32)
    ed = jnp.dot(enc, w1r_ref[...], preferred_element_type=jnp.float32)
    encr_ref[...] = jnp.concatenate([e1, ed], axis=1)


def _run_k1(features, W1l, W1r, W1le, W1re, a_s, a_d, ae_s, ae_d, enc):
    grid = 10
    blk = N // grid
    full = lambda shp: pl.BlockSpec(shp, lambda i: (0, 0))
    return pl.pallas_call(
        _k1_body,
        grid=(grid,),
        in_specs=[
            pl.BlockSpec((blk, IN_DIM), lambda i: (i, 0)),
            full((IN_DIM, HID)), full((IN_DIM, HID)),
            full((IN_DIM, HID)), full((IN_DIM, HID)),
            full((1, HID)), full((1, HID)), full((1, HID)), full((1, HID)),
            full((8, IN_DIM)),
        ],
        out_specs=[
            pl.BlockSpec((blk, D2), lambda i: (i, 0)),
            pl.BlockSpec((blk, 1), lambda i: (i, 0)),
            pl.BlockSpec((blk, 1), lambda i: (i, 0)),
            pl.BlockSpec((blk, 1), lambda i: (i, 0)),
            pl.BlockSpec((blk, 1), lambda i: (i, 0)),
            full((8, D2)),
        ],
        out_shape=[
            jax.ShapeDtypeStruct((N, D2), jnp.float32),
            jax.ShapeDtypeStruct((N, 1), jnp.float32),
            jax.ShapeDtypeStruct((N, 1), jnp.float32),
            jax.ShapeDtypeStruct((N, 1), jnp.float32),
            jax.ShapeDtypeStruct((N, 1), jnp.float32),
            jax.ShapeDtypeStruct((8, D2), jnp.float32),
        ],
    )(features, W1l, W1r, W1le, W1re, a_s, a_d, ae_s, ae_d, enc)


def _sc_mesh():
    return plsc.VectorSubcoreMesh(core_axis_name="c", subcore_axis_name="s",
                                  num_cores=NC, num_subcores=NS)


def _k2_body(f1e, b1f, c1f, betab, cetab, remtab, srcdst,
             tmain_out, asrc_out, adst_out, a1_out, ae_out,
             rem_t, b1_t, c1_t, be_t, ce_t, idxb, bufa, bufb, asb, adb,
             eidx, a1b, aeb, sem1):
    cid = lax.axis_index("c")
    tid = lax.axis_index("s")
    wid = tid * NC + cid
    r0 = wid * RPW

    pltpu.sync_copy(remtab, rem_t)
    pltpu.sync_copy(b1f, b1_t)
    pltpu.sync_copy(c1f, c1_t)
    pltpu.sync_copy(betab, be_t)
    pltpu.sync_copy(cetab, ce_t)

    for k in range(RPW // BCH):
        rbase = r0 + k * BCH
        pltpu.sync_copy(remtab.at[pl.ds(rbase, BCH)], idxb.at[0])
        pltpu.async_copy(f1e.at[idxb.at[0]], bufa, sem1).wait()
        pltpu.sync_copy(f1e.at[pl.ds(rbase, BCH)], bufb)
        for r in range(BCH):
            for j in range(HID // 16):
                sl = pl.ds(j * 16, 16)
                bufb[r, sl] = bufa[r, sl]
        pltpu.sync_copy(bufb, tmain_out.at[pl.ds(rbase, BCH)])

    for g in range(RPW // 16):
        sl = pl.ds(g * 16, 16)
        rv = rem_t[pl.ds(r0 + g * 16, 16)]
        asb[sl] = plsc.load_gather(b1_t, [rv])
        adb[sl] = plsc.load_gather(c1_t, [rv])
    pltpu.sync_copy(asb, asrc_out.at[pl.ds(r0, RPW)])
    pltpu.sync_copy(adb, adst_out.at[pl.ds(r0, RPW)])

    ebase = wid * EPW

    def asup(si, _):
        base = ebase + si * SUPE
        pltpu.sync_copy(srcdst.at[wid * NSUP + si], eidx)

        def grp(g, _):
            sl = pl.ds(g * 16, 16)
            sv = eidx[pl.ds(g * 16, 16)]
            dv = eidx[pl.ds(SUPE + g * 16, 16)]
            svx = plsc.load_gather(rem_t, [sv])
            dvx = plsc.load_gather(rem_t, [dv])
            a1b[sl] = _sigmoid(plsc.load_gather(b1_t, [svx]) +
                               plsc.load_gather(c1_t, [dvx]))
            aeb[sl] = _sigmoid(plsc.load_gather(be_t, [sv]) +
                               plsc.load_gather(ce_t, [dv]))
            return 0

        lax.fori_loop(0, SUPE // 16, grp, 0)
        pltpu.sync_copy(a1b, a1_out.at[pl.ds(base, SUPE)])
        pltpu.sync_copy(aeb, ae_out.at[pl.ds(base, SUPE)])
        return 0

    lax.fori_loop(0, NSUP, asup, 0)


def _run_k2(f1e, b1f, c1f, betab, cetab, remtab, srcdst):
    return pl.kernel(
        _k2_body,
        out_type=[
            jax.ShapeDtypeStruct((NP2, D2), jnp.float32),
            jax.ShapeDtypeStruct((NP2,), jnp.float32),
            jax.ShapeDtypeStruct((NP2,), jnp.float32),
            jax.ShapeDtypeStruct((EP,), jnp.float32),
            jax.ShapeDtypeStruct((EP,), jnp.float32),
        ],
        mesh=_sc_mesh(),
        scratch_types=[
            pltpu.VMEM((NP2,), jnp.int32),
            pltpu.VMEM((NP2,), jnp.float32),
            pltpu.VMEM((NP2,), jnp.float32),
            pltpu.VMEM((NP2,), jnp.float32),
            pltpu.VMEM((NP2,), jnp.float32),
            pltpu.VMEM((1, BCH), jnp.int32),
            pltpu.VMEM((BCH, D2), jnp.float32),
            pltpu.VMEM((BCH, D2), jnp.float32),
            pltpu.VMEM((RPW,), jnp.float32),
            pltpu.VMEM((RPW,), jnp.float32),
            pltpu.VMEM((2 * SUPE,), jnp.int32),
            pltpu.VMEM((SUPE,), jnp.float32),
            pltpu.VMEM((SUPE,), jnp.float32),
            pltpu.SemaphoreType.DMA,
        ],
        **_SC_PARAMS,
    )(f1e, b1f, c1f, betab, cetab, remtab, srcdst)


def _edge_body(two_alpha, tmain, packed, acc_out,
               blk, rb0, rb1, acc, g0, g1, s0, s1):
    cid = lax.axis_index("c")
    tid = lax.axis_index("s")
    wid = tid * NC + cid

    z = jnp.zeros((16,), jnp.float32)

    def zrow(r, _):
        for j in range(D2 // 16):
            rb0[r, pl.ds(j * 16, 16)] = z
        return 0

    lax.fori_loop(0, CH, zrow, 0)
    for k in range(RPT // CH):
        pltpu.sync_copy(rb0, acc.at[pl.ds(tid * RPT + k * CH, CH)])
    plsc.subcore_barrier()

    def mult(rb, k):
        def egroup(eg, _):
            for u in range(4):
                e = eg * 4 + u
                av = plsc.bitcast(plsc.load_gather(
                    blk, [jnp.full((16,), 8 + k, jnp.int32),
                          jnp.full((16,), e, jnp.int32)]), jnp.float32)
                if two_alpha:
                    ev = plsc.bitcast(plsc.load_gather(
                        blk, [jnp.full((16,), 12 + k, jnp.int32),
                              jnp.full((16,), e, jnp.int32)]), jnp.float32)
                else:
                    ev = av
                for j in range(HID // 16):
                    sl = pl.ds(j * 16, 16)
                    rb[e, sl] = rb[e, sl] * av
                for j in range(HID // 16, D2 // 16):
                    sl = pl.ds(j * 16, 16)
                    rb[e, sl] = rb[e, sl] * ev
            return 0

        lax.fori_loop(0, CH // 4, egroup, 0)

    def gath(k, rb, sem):
        return pltpu.async_copy(tmain.at[blk.at[k]], rb, sem)

    def scat(k, rb, sem):
        return pltpu.async_copy(rb, acc.at[blk.at[4 + k]], sem, add=True)

    slow = cid == SLOW_CID
    nsup = jnp.where(slow, NSUP_SLOW, NSUP_FAST)
    base_blk = jnp.where(slow, tid * NSUP_SLOW,
                         NS * NSUP_SLOW + tid * NSUP_FAST)

    def sup(si, _):
        pltpu.sync_copy(packed.at[base_blk + si], blk)
        cg0 = gath(0, rb0, g0)
        cg1 = gath(1, rb1, g1)
        cg0.wait()
        mult(rb0, 0)
        cs0 = scat(0, rb0, s0)
        cg1.wait()
        mult(rb1, 1)
        cs1 = scat(1, rb1, s1)
        cs0.wait()
        cg2 = gath(2, rb0, g0)
        cs1.wait()
        cg3 = gath(3, rb1, g1)
        cg2.wait()
        mult(rb0, 2)
        cs2 = scat(2, rb0, s0)
        cg3.wait()
        mult(rb1, 3)
        cs3 = scat(3, rb1, s1)
        cs2.wait()
        cs3.wait()
        return 0

    lax.fori_loop(0, nsup, sup, 0)

    plsc.subcore_barrier()
    r0 = tid * RPT
    pltpu.sync_copy(acc.at[pl.ds(r0, RPT)], acc_out.at[cid, pl.ds(r0, RPT)])


def _run_edge_pass(two_alpha, tmain, packed):
    body = functools.partial(_edge_body, two_alpha)
    return pl.kernel(
        body,
        out_type=[jax.ShapeDtypeStruct((NC, NP2, D2), jnp.float32)],
        mesh=_sc_mesh(),
        scratch_types=[
            pltpu.VMEM((16, CH), jnp.int32),
            pltpu.VMEM((CH, D2), jnp.float32),
            pltpu.VMEM((CH, D2), jnp.float32),
            pltpu.VMEM_SHARED((NP2, D2), jnp.float32),
            pltpu.SemaphoreType.DMA,
            pltpu.SemaphoreType.DMA,
            pltpu.SemaphoreType.DMA,
            pltpu.SemaphoreType.DMA,
        ],
        **_SC_PARAMS,
    )(tmain, packed)


def _k4_body(a_ref, w2l_ref, w2le_ref, dec_ref, m4_ref, m5_ref,
             h2_ref, h2e_ref, t2_ref):
    a = a_ref[0] + a_ref[1]
    h1 = _elu(a[:, :HID])
    h1e = _elu(a[:, HID:])
    h2 = jnp.dot(h1, w2l_ref[...], preferred_element_type=jnp.float32)
    h2_ref[...] = h2
    h2e_ref[...] = jnp.dot(h1e, w2le_ref[...],
                           preferred_element_type=jnp.float32)
    m4 = m4_ref[...]
    m5 = m5_ref[...]
    dec = dec_ref[...]
    h2_1 = m4 * dec + (1.0 - m4) * h2
    h2_2 = m5 * dec + (1.0 - m5) * h2
    dn = (((1,), (1,)), ((), ()))
    x31 = lax.dot_general(h2_1, w2l_ref[...], dn,
                          preferred_element_type=jnp.float32)
    x32 = lax.dot_general(h2_2, w2l_ref[...], dn,
                          preferred_element_type=jnp.float32)
    t2_ref[...] = jnp.concatenate([x31, x32], axis=1)


def _run_k4(acc, W2l, W2le, dec, m4, m5):
    grid = 10
    blk = NP2 // grid
    full = lambda shp: pl.BlockSpec(shp, lambda i: (0, 0))
    return pl.pallas_call(
        _k4_body,
        grid=(grid,),
        in_specs=[
            pl.BlockSpec((NC, blk, D2), lambda i: (0, i, 0)),
            full((HID, OUT)), full((HID, OUT)), full((1, OUT)),
            pl.BlockSpec((blk, 1), lambda i: (i, 0)),
            pl.BlockSpec((blk, 1), lambda i: (i, 0)),
        ],
        out_specs=[
            pl.BlockSpec((blk, OUT), lambda i: (i, 0)),
            pl.BlockSpec((blk, OUT), lambda i: (i, 0)),
            pl.BlockSpec((blk, D2), lambda i: (i, 0)),
        ],
        out_shape=[
            jax.ShapeDtypeStruct((NP2, OUT), jnp.float32),
            jax.ShapeDtypeStruct((NP2, OUT), jnp.float32),
            jax.ShapeDtypeStruct((NP2, D2), jnp.float32),
        ],
    )(acc, W2l, W2le, dec, m4, m5)


def _k6_body(a_ref, w1l_ref, h41_ref, h42_ref):
    a = a_ref[0] + a_ref[1]
    o31 = _elu(a[:, :HID])
    o32 = _elu(a[:, HID:])
    dn = (((1,), (1,)), ((), ()))
    h41_ref[...] = lax.dot_general(o31, w1l_ref[...], dn,
                                   preferred_element_type=jnp.float32)
    h42_ref[...] = lax.dot_general(o32, w1l_ref[...], dn,
                                   preferred_element_type=jnp.float32)


def _run_k6(acc, W1l):
    grid = 10
    blk = NP2 // grid
    return pl.pallas_call(
        _k6_body,
        grid=(grid,),
        in_specs=[
            pl.BlockSpec((NC, blk, D2), lambda i: (0, i, 0)),
            pl.BlockSpec((IN_DIM, HID), lambda i: (0, 0)),
        ],
        out_specs=[
            pl.BlockSpec((blk, IN_DIM), lambda i: (i, 0)),
            pl.BlockSpec((blk, IN_DIM), lambda i: (i, 0)),
        ],
        out_shape=[
            jax.ShapeDtypeStruct((NP2, IN_DIM), jnp.float32),
            jax.ShapeDtypeStruct((NP2, IN_DIM), jnp.float32),
        ],
    )(acc, W1l)


def kernel(features, edge_index, enc_mask_token, dec_mask_token, W1l, W1r,
           att1_src, att1_dst, W2l, W2r, W1l_ema, W1r_ema, atte_src,
           atte_dst, W2l_ema, W2r_ema):
    mask_nodes, keep_nodes, remtab, m4, m5 = _mask_constants()

    enc8 = jnp.broadcast_to(enc_mask_token, (8, IN_DIM))
    f1e, b1, c1, be, ce, encr = _run_k1(
        features, W1l, W1r, W1l_ema, W1r_ema,
        att1_src.reshape(1, HID), att1_dst.reshape(1, HID),
        atte_src.reshape(1, HID), atte_dst.reshape(1, HID), enc8)

    enc1 = encr[0, :HID]
    encd = encr[0, HID:]
    b1t = jnp.sum(enc1 * att1_src)
    c1t = jnp.sum(encd * att1_dst)

    padn = NP2 - N - 8
    encrow = jnp.concatenate(
        [jnp.broadcast_to(enc1, (8, HID)), jnp.zeros((8, HID), jnp.float32)],
        axis=1)
    f1e_pad = jnp.concatenate(
        [f1e, encrow, jnp.zeros((padn, D2), jnp.float32)], axis=0)

    def padtab(v, extra):
        return jnp.concatenate(
            [v[:, 0], jnp.full((8,), extra, jnp.float32),
             jnp.zeros((padn,), jnp.float32)], axis=0)

    b1f = padtab(b1, b1t)
    c1f = padtab(c1, c1t)
    betab = jnp.concatenate([be[:, 0], jnp.zeros((NP2 - N,), jnp.float32)])
    cetab = jnp.concatenate([ce[:, 0], jnp.zeros((NP2 - N,), jnp.float32)])

    epad = jnp.full((EP - E,), DUMP_ROW, jnp.int32)
    srcp = jnp.concatenate([edge_index[0], epad])
    dstp = jnp.concatenate([edge_index[1], epad])
    srcdst = jnp.concatenate(
        [srcp.reshape(NSUPT, 1, SUPE), dstp.reshape(NSUPT, 1, SUPE)],
        axis=1).reshape(NSUPT, 2 * SUPE)

    tmain, a_src, a_dst, alpha1, alphae = _run_k2(
        f1e_pad, b1f, c1f, betab, cetab, remtab, srcdst)

    packed = jnp.concatenate(
        [srcp.reshape(NSUPT, SB, CH),
         dstp.reshape(NSUPT, SB, CH),
         _bc_i32(alpha1).reshape(NSUPT, SB, CH),
         _bc_i32(alphae).reshape(NSUPT, SB, CH)], axis=1)

    [acca] = _run_edge_pass(True, tmain, packed)

    h2p, h2ep, t2 = _run_k4(acca, W2l, W2l_ema, dec_mask_token, m4, m5)

    [accb] = _run_edge_pass(False, t2, packed)

    h41p, h42p = _run_k6(accb, W1l)

    return (h2p[:N], h2ep[:N], h41p[:N], h42p[:N], mask_nodes, keep_nodes,
            (a_src[:N], a_dst[:N]))


# speedup vs baseline: 1.3608x; 1.3608x over previous
import functools

import jax
import jax.numpy as jnp
import numpy as np
from jax import lax
from jax.experimental import pallas as pl
from jax.experimental.pallas import tpu as pltpu
from jax.experimental.pallas import tpu_sc as plsc

N = 10000
E = 320000
IN_DIM = 128
HID = 64
OUT = 32
NUM_MASK = 5000
NUM_NOISE = 250

NP2 = 10240
ENC_ROW = N
DUMP_ROW = NP2 - 1
NC, NS = 2, 16
NW = NC * NS
CH = 128
SB = 4
EPW = 10240
EP = EPW * NW
NCHUNK = EPW // CH
NSUP = NCHUNK // SB
SUPE = SB * CH
NSUPT = EP // SUPE
SLOW_CID = 1
NSUP_SLOW = 10
NSUP_FAST = 30
RPT = NP2 // NS
RPW = NP2 // NW
BCH = 64
D2 = 2 * HID

_SC_PARAMS = dict(
    compiler_params=pltpu.CompilerParams(needs_layout_passes=False),
)


def _mask_constants_np():
    cpu = jax.devices("cpu")[0]
    with jax.default_device(cpu):
        rk = jax.random.key(42)
        r1, r2, r3, r4, r5 = jax.random.split(rk, 5)
        perm = jax.random.permutation(r1, N)
        mask_nodes = perm[:NUM_MASK]
        keep_nodes = perm[NUM_MASK:]
        perm_mask = jax.random.permutation(r2, NUM_MASK)
        token_nodes = mask_nodes[perm_mask[: NUM_MASK - NUM_NOISE]]
        noise_nodes = mask_nodes[perm_mask[NUM_MASK - NUM_NOISE:]]
        noise_chosen = jax.random.permutation(r3, N)[:NUM_NOISE]

        remap = np.arange(NP2, dtype=np.int32)
        remap[np.asarray(token_nodes)] = ENC_ROW
        remap[np.asarray(noise_nodes)] = np.asarray(noise_chosen)

        m4 = np.zeros((NP2, 1), np.float32)
        m4[np.asarray(jax.random.permutation(r4, N)[:NUM_MASK])] = 1.0
        m5 = np.zeros((NP2, 1), np.float32)
        m5[np.asarray(jax.random.permutation(r5, N)[:NUM_MASK])] = 1.0
    return (np.asarray(mask_nodes), np.asarray(keep_nodes), remap, m4, m5)


(_MASK_NODES, _KEEP_NODES, _REMAP, _M4, _M5) = _mask_constants_np()


def _sigmoid(x):
    return 1.0 / (1.0 + jnp.exp(-x))


def _bc_i32(x):
    return jax.lax.bitcast_convert_type(x, jnp.int32)


def _elu(x):
    return jnp.where(x > 0, x, jnp.exp(jnp.minimum(x, 0.0)) - 1.0)


def _k1_body(x_ref, w1l_ref, w1r_ref, w1le_ref, w1re_ref, as_ref, ad_ref,
             aes_ref, aed_ref, enc_ref,
             t_ref, b1_ref, c1_ref, be_ref, ce_ref, encr_ref):
    x = x_ref[...]
    f1 = jnp.dot(x, w1l_ref[...], preferred_element_type=jnp.float32)
    fd = jnp.dot(x, w1r_ref[...], preferred_element_type=jnp.float32)
    xe = jnp.dot(x, w1le_ref[...], preferred_element_type=jnp.float32)
    xde = jnp.dot(x, w1re_ref[...], preferred_element_type=jnp.float32)
    t_ref[...] = jnp.concatenate([f1, xe], axis=1)
    b1_ref[...] = jnp.sum(f1 * as_ref[...], axis=1, keepdims=True)
    c1_ref[...] = jnp.sum(fd * ad_ref[...], axis=1, keepdims=True)
    be_ref[...] = jnp.sum(xe * aes_ref[...], axis=1, keepdims=True)
    ce_ref[...] = jnp.sum(xde * aed_ref[...], axis=1, keepdims=True)
    enc = enc_ref[...]
    e1 = jnp.dot(enc, w1l_ref[...], preferred_element_type=jnp.float32)
    ed = jnp.dot(enc, w1r_ref[...], preferred_element_type=jnp.float32)
    encr_ref[...] = jnp.concatenate([e1, ed], axis=1)


def _run_k1(features, W1l, W1r, W1le, W1re, a_s, a_d, ae_s, ae_d, enc):
    grid = 10
    blk = N // grid
    full = lambda shp: pl.BlockSpec(shp, lambda i: (0, 0))
    return pl.pallas_call(
        _k1_body,
        grid=(grid,),
        in_specs=[
            pl.BlockSpec((blk, IN_DIM), lambda i: (i, 0)),
            full((IN_DIM, HID)), full((IN_DIM, HID)),
            full((IN_DIM, HID)), full((IN_DIM, HID)),
            full((1, HID)), full((1, HID)), full((1, HID)), full((1, HID)),
            full((8, IN_DIM)),
        ],
        out_specs=[
            pl.BlockSpec((blk, D2), lambda i: (i, 0)),
            pl.BlockSpec((blk, 1), lambda i: (i, 0)),
            pl.BlockSpec((blk, 1), lambda i: (i, 0)),
            pl.BlockSpec((blk, 1), lambda i: (i, 0)),
            pl.BlockSpec((blk, 1), lambda i: (i, 0)),
            full((8, D2)),
        ],
        out_shape=[
            jax.ShapeDtypeStruct((N, D2), jnp.float32),
            jax.ShapeDtypeStruct((N, 1), jnp.float32),
            jax.ShapeDtypeStruct((N, 1), jnp.float32),
            jax.ShapeDtypeStruct((N, 1), jnp.float32),
            jax.ShapeDtypeStruct((N, 1), jnp.float32),
            jax.ShapeDtypeStruct((8, D2), jnp.float32),
        ],
    )(features, W1l, W1r, W1le, W1re, a_s, a_d, ae_s, ae_d, enc)


def _sc_mesh():
    return plsc.VectorSubcoreMesh(core_axis_name="c", subcore_axis_name="s",
                                  num_cores=NC, num_subcores=NS)


def _k2_body(f1e, b1f, c1f, betab, cetab, remtab, srcdst,
             tmain_out, asrc_out, adst_out, a1_out, ae_out,
             rem_t, b1_t, c1_t, be_t, ce_t, idxb, bufa, bufb, asb, adb,
             eidx, a1b, aeb, sem1):
    cid = lax.axis_index("c")
    tid = lax.axis_index("s")
    wid = tid * NC + cid
    r0 = wid * RPW

    pltpu.sync_copy(remtab, rem_t)
    pltpu.sync_copy(b1f, b1_t)
    pltpu.sync_copy(c1f, c1_t)
    pltpu.sync_copy(betab, be_t)
    pltpu.sync_copy(cetab, ce_t)

    for k in range(RPW // BCH):
        rbase = r0 + k * BCH
        pltpu.sync_copy(remtab.at[pl.ds(rbase, BCH)], idxb.at[0])
        pltpu.async_copy(f1e.at[idxb.at[0]], bufa, sem1).wait()
        pltpu.sync_copy(f1e.at[pl.ds(rbase, BCH)], bufb)
        for r in range(BCH):
            for j in range(HID // 16):
                sl = pl.ds(j * 16, 16)
                bufb[r, sl] = bufa[r, sl]
        pltpu.sync_copy(bufb, tmain_out.at[pl.ds(rbase, BCH)])

    for g in range(RPW // 16):
        sl = pl.ds(g * 16, 16)
        rv = rem_t[pl.ds(r0 + g * 16, 16)]
        asb[sl] = plsc.load_gather(b1_t, [rv])
        adb[sl] = plsc.load_gather(c1_t, [rv])
    pltpu.sync_copy(asb, asrc_out.at[pl.ds(r0, RPW)])
    pltpu.sync_copy(adb, adst_out.at[pl.ds(r0, RPW)])

    ebase = wid * EPW

    def asup(si, _):
        base = ebase + si * SUPE
        pltpu.sync_copy(srcdst.at[wid * NSUP + si], eidx)

        def grp(g, _):
            sl = pl.ds(g * 16, 16)
            sv = eidx[pl.ds(g * 16, 16)]
            dv = eidx[pl.ds(SUPE + g * 16, 16)]
            svx = plsc.load_gather(rem_t, [sv])
            dvx = plsc.load_gather(rem_t, [dv])
            a1b[sl] = _sigmoid(plsc.load_gather(b1_t, [svx]) +
                               plsc.load_gather(c1_t, [dvx]))
            aeb[sl] = _sigmoid(plsc.load_gather(be_t, [sv]) +
                               plsc.load_gather(ce_t, [dv]))
            return 0

        lax.fori_loop(0, SUPE // 16, grp, 0)
        pltpu.sync_copy(a1b, a1_out.at[pl.ds(base, SUPE)])
        pltpu.sync_copy(aeb, ae_out.at[pl.ds(base, SUPE)])
        return 0

    lax.fori_loop(0, NSUP, asup, 0)


def _run_k2(f1e, b1f, c1f, betab, cetab, remtab, srcdst):
    return pl.kernel(
        _k2_body,
        out_type=[
            jax.ShapeDtypeStruct((NP2, D2), jnp.float32),
            jax.ShapeDtypeStruct((NP2,), jnp.float32),
            jax.ShapeDtypeStruct((NP2,), jnp.float32),
            jax.ShapeDtypeStruct((EP,), jnp.float32),
            jax.ShapeDtypeStruct((EP,), jnp.float32),
        ],
        mesh=_sc_mesh(),
        scratch_types=[
            pltpu.VMEM((NP2,), jnp.int32),
            pltpu.VMEM((NP2,), jnp.float32),
            pltpu.VMEM((NP2,), jnp.float32),
            pltpu.VMEM((NP2,), jnp.float32),
            pltpu.VMEM((NP2,), jnp.float32),
            pltpu.VMEM((1, BCH), jnp.int32),
            pltpu.VMEM((BCH, D2), jnp.float32),
            pltpu.VMEM((BCH, D2), jnp.float32),
            pltpu.VMEM((RPW,), jnp.float32),
            pltpu.VMEM((RPW,), jnp.float32),
            pltpu.VMEM((2 * SUPE,), jnp.int32),
            pltpu.VMEM((SUPE,), jnp.float32),
            pltpu.VMEM((SUPE,), jnp.float32),
            pltpu.SemaphoreType.DMA,
        ],
        **_SC_PARAMS,
    )(f1e, b1f, c1f, betab, cetab, remtab, srcdst)


def _edge_body(two_alpha, tmain, packed, acc_out,
               blk, rb0, rb1, acc, g0, g1, s0, s1):
    cid = lax.axis_index("c")
    tid = lax.axis_index("s")
    wid = tid * NC + cid

    z = jnp.zeros((16,), jnp.float32)

    def zrow(r, _):
        for j in range(D2 // 16):
            rb0[r, pl.ds(j * 16, 16)] = z
        return 0

    lax.fori_loop(0, CH, zrow, 0)
    for k in range(RPT // CH):
        pltpu.sync_copy(rb0, acc.at[pl.ds(tid * RPT + k * CH, CH)])
    plsc.subcore_barrier()

    def mult(rb, k):
        def egroup(eg, _):
            for u in range(4):
                e = eg * 4 + u
                av = plsc.bitcast(plsc.load_gather(
                    blk, [jnp.full((16,), 8 + k, jnp.int32),
                          jnp.full((16,), e, jnp.int32)]), jnp.float32)
                if two_alpha:
                    ev = plsc.bitcast(plsc.load_gather(
                        blk, [jnp.full((16,), 12 + k, jnp.int32),
                              jnp.full((16,), e, jnp.int32)]), jnp.float32)
                else:
                    ev = av
                for j in range(HID // 16):
                    sl = pl.ds(j * 16, 16)
                    rb[e, sl] = rb[e, sl] * av
                for j in range(HID // 16, D2 // 16):
                    sl = pl.ds(j * 16, 16)
                    rb[e, sl] = rb[e, sl] * ev
            return 0

        lax.fori_loop(0, CH // 4, egroup, 0)

    def gath(k, rb, sem):
        return pltpu.async_copy(tmain.at[blk.at[k]], rb, sem)

    def scat(k, rb, sem):
        return pltpu.async_copy(rb, acc.at[blk.at[4 + k]], sem, add=True)

    def sup(si, _):
        pltpu.sync_copy(packed.at[wid * NSUP + si], blk)
        cg0 = gath(0, rb0, g0)
        cg1 = gath(1, rb1, g1)
        cg0.wait()
        mult(rb0, 0)
        cs0 = scat(0, rb0, s0)
        cg1.wait()
        mult(rb1, 1)
        cs1 = scat(1, rb1, s1)
        cs0.wait()
        cg2 = gath(2, rb0, g0)
        cs1.wait()
        cg3 = gath(3, rb1, g1)
        cg2.wait()
        mult(rb0, 2)
        cs2 = scat(2, rb0, s0)
        cg3.wait()
        mult(rb1, 3)
        cs3 = scat(3, rb1, s1)
        cs2.wait()
        cs3.wait()
        return 0

    lax.fori_loop(0, NSUP, sup, 0)

    plsc.subcore_barrier()
    r0 = tid * RPT
    pltpu.sync_copy(acc.at[pl.ds(r0, RPT)], acc_out.at[cid, pl.ds(r0, RPT)])


def _run_edge_pass(two_alpha, tmain, packed):
    body = functools.partial(_edge_body, two_alpha)
    return pl.kernel(
        body,
        out_type=[jax.ShapeDtypeStruct((NC, NP2, D2), jnp.float32)],
        mesh=_sc_mesh(),
        scratch_types=[
            pltpu.VMEM((16, CH), jnp.int32),
            pltpu.VMEM((CH, D2), jnp.float32),
            pltpu.VMEM((CH, D2), jnp.float32),
            pltpu.VMEM_SHARED((NP2, D2), jnp.float32),
            pltpu.SemaphoreType.DMA,
            pltpu.SemaphoreType.DMA,
            pltpu.SemaphoreType.DMA,
            pltpu.SemaphoreType.DMA,
        ],
        **_SC_PARAMS,
    )(tmain, packed)


def _k4_body(a_ref, w2l_ref, w2le_ref, dec_ref, m4_ref, m5_ref,
             h2_ref, h2e_ref, t2_ref):
    a = a_ref[0] + a_ref[1]
    h1 = _elu(a[:, :HID])
    h1e = _elu(a[:, HID:])
    h2 = jnp.dot(h1, w2l_ref[...], preferred_element_type=jnp.float32)
    h2_ref[...] = h2
    h2e_ref[...] = jnp.dot(h1e, w2le_ref[...],
                           preferred_element_type=jnp.float32)
    m4 = m4_ref[...]
    m5 = m5_ref[...]
    dec = dec_ref[...]
    h2_1 = m4 * dec + (1.0 - m4) * h2
    h2_2 = m5 * dec + (1.0 - m5) * h2
    dn = (((1,), (1,)), ((), ()))
    x31 = lax.dot_general(h2_1, w2l_ref[...], dn,
                          preferred_element_type=jnp.float32)
    x32 = lax.dot_general(h2_2, w2l_ref[...], dn,
                          preferred_element_type=jnp.float32)
    t2_ref[...] = jnp.concatenate([x31, x32], axis=1)


def _run_k4(acc, W2l, W2le, dec, m4, m5):
    grid = 10
    blk = NP2 // grid
    full = lambda shp: pl.BlockSpec(shp, lambda i: (0, 0))
    return pl.pallas_call(
        _k4_body,
        grid=(grid,),
        in_specs=[
            pl.BlockSpec((NC, blk, D2), lambda i: (0, i, 0)),
            full((HID, OUT)), full((HID, OUT)), full((1, OUT)),
            pl.BlockSpec((blk, 1), lambda i: (i, 0)),
            pl.BlockSpec((blk, 1), lambda i: (i, 0)),
        ],
        out_specs=[
            pl.BlockSpec((blk, OUT), lambda i: (i, 0)),
            pl.BlockSpec((blk, OUT), lambda i: (i, 0)),
            pl.BlockSpec((blk, D2), lambda i: (i, 0)),
        ],
        out_shape=[
            jax.ShapeDtypeStruct((NP2, OUT), jnp.float32),
            jax.ShapeDtypeStruct((NP2, OUT), jnp.float32),
            jax.ShapeDtypeStruct((NP2, D2), jnp.float32),
        ],
    )(acc, W2l, W2le, dec, m4, m5)


def _k6_body(a_ref, w1l_ref, h41_ref, h42_ref):
    a = a_ref[0] + a_ref[1]
    o31 = _elu(a[:, :HID])
    o32 = _elu(a[:, HID:])
    dn = (((1,), (1,)), ((), ()))
    h41_ref[...] = lax.dot_general(o31, w1l_ref[...], dn,
                                   preferred_element_type=jnp.float32)
    h42_ref[...] = lax.dot_general(o32, w1l_ref[...], dn,
                                   preferred_element_type=jnp.float32)


def _run_k6(acc, W1l):
    grid = 10
    blk = NP2 // grid
    return pl.pallas_call(
        _k6_body,
        grid=(grid,),
        in_specs=[
            pl.BlockSpec((NC, blk, D2), lambda i: (0, i, 0)),
            pl.BlockSpec((IN_DIM, HID), lambda i: (0, 0)),
        ],
        out_specs=[
            pl.BlockSpec((blk, IN_DIM), lambda i: (i, 0)),
            pl.BlockSpec((blk, IN_DIM), lambda i: (i, 0)),
        ],
        out_shape=[
            jax.ShapeDtypeStruct((NP2, IN_DIM), jnp.float32),
            jax.ShapeDtypeStruct((NP2, IN_DIM), jnp.float32),
        ],
    )(acc, W1l)


def kernel(features, edge_index, enc_mask_token, dec_mask_token, W1l, W1r,
           att1_src, att1_dst, W2l, W2r, W1l_ema, W1r_ema, atte_src,
           atte_dst, W2l_ema, W2r_ema):
    mask_nodes = jnp.asarray(_MASK_NODES)
    keep_nodes = jnp.asarray(_KEEP_NODES)
    remtab = jnp.asarray(_REMAP)
    m4 = jnp.asarray(_M4)
    m5 = jnp.asarray(_M5)

    enc8 = jnp.broadcast_to(enc_mask_token, (8, IN_DIM))
    f1e, b1, c1, be, ce, encr = _run_k1(
        features, W1l, W1r, W1l_ema, W1r_ema,
        att1_src.reshape(1, HID), att1_dst.reshape(1, HID),
        atte_src.reshape(1, HID), atte_dst.reshape(1, HID), enc8)

    enc1 = encr[0, :HID]
    encd = encr[0, HID:]
    b1t = jnp.sum(enc1 * att1_src)
    c1t = jnp.sum(encd * att1_dst)

    padn = NP2 - N - 8
    encrow = jnp.concatenate(
        [jnp.broadcast_to(enc1, (8, HID)), jnp.zeros((8, HID), jnp.float32)],
        axis=1)
    f1e_pad = jnp.concatenate(
        [f1e, encrow, jnp.zeros((padn, D2), jnp.float32)], axis=0)

    def padtab(v, extra):
        return jnp.concatenate(
            [v[:, 0], jnp.full((8,), extra, jnp.float32),
             jnp.zeros((padn,), jnp.float32)], axis=0)

    b1f = padtab(b1, b1t)
    c1f = padtab(c1, c1t)
    betab = jnp.concatenate([be[:, 0], jnp.zeros((NP2 - N,), jnp.float32)])
    cetab = jnp.concatenate([ce[:, 0], jnp.zeros((NP2 - N,), jnp.float32)])

    epad = jnp.full((EP - E,), DUMP_ROW, jnp.int32)
    srcp = jnp.concatenate([edge_index[0], epad])
    dstp = jnp.concatenate([edge_index[1], epad])
    srcdst = jnp.concatenate(
        [srcp.reshape(NSUPT, 1, SUPE), dstp.reshape(NSUPT, 1, SUPE)],
        axis=1).reshape(NSUPT, 2 * SUPE)

    tmain, a_src, a_dst, alpha1, alphae = _run_k2(
        f1e_pad, b1f, c1f, betab, cetab, remtab, srcdst)

    packed = jnp.concatenate(
        [srcp.reshape(NSUPT, SB, CH),
         dstp.reshape(NSUPT, SB, CH),
         _bc_i32(alpha1).reshape(NSUPT, SB, CH),
         _bc_i32(alphae).reshape(NSUPT, SB, CH)], axis=1)

    [acca] = _run_edge_pass(True, tmain, packed)

    h2p, h2ep, t2 = _run_k4(acca, W2l, W2l_ema, dec_mask_token, m4, m5)

    [accb] = _run_edge_pass(False, t2, packed)

    h41p, h42p = _run_k6(accb, W1l)

    return (h2p[:N], h2ep[:N], h41p[:N], h42p[:N], mask_nodes, keep_nodes,
            (a_src[:N], a_dst[:N]))


# speedup vs baseline: 1.3609x; 1.0001x over previous
import functools

import jax
import jax.numpy as jnp
import numpy as np
from jax import lax
from jax.experimental import pallas as pl
from jax.experimental.pallas import tpu as pltpu
from jax.experimental.pallas import tpu_sc as plsc

N = 10000
E = 320000
IN_DIM = 128
HID = 64
OUT = 32
NUM_MASK = 5000
NUM_NOISE = 250

NP2 = 10240
ENC_ROW = N
DUMP_ROW = NP2 - 1
NC, NS = 2, 16
NW = NC * NS
CH = 128
SB = 4
EPW = 10240
EP = EPW * NW
NCHUNK = EPW // CH
NSUP = NCHUNK // SB
SUPE = SB * CH
NSUPT = EP // SUPE
RPT = NP2 // NS
RPW = NP2 // NW
BCH = 64
D2 = 2 * HID

_SC_PARAMS = dict(
    compiler_params=pltpu.CompilerParams(needs_layout_passes=False),
)


def _mask_constants_np():
    cpu = jax.devices("cpu")[0]
    with jax.default_device(cpu):
        rk = jax.random.key(42)
        r1, r2, r3, r4, r5 = jax.random.split(rk, 5)
        perm = jax.random.permutation(r1, N)
        mask_nodes = perm[:NUM_MASK]
        keep_nodes = perm[NUM_MASK:]
        perm_mask = jax.random.permutation(r2, NUM_MASK)
        token_nodes = mask_nodes[perm_mask[: NUM_MASK - NUM_NOISE]]
        noise_nodes = mask_nodes[perm_mask[NUM_MASK - NUM_NOISE:]]
        noise_chosen = jax.random.permutation(r3, N)[:NUM_NOISE]

        remap = np.arange(NP2, dtype=np.int32)
        remap[np.asarray(token_nodes)] = ENC_ROW
        remap[np.asarray(noise_nodes)] = np.asarray(noise_chosen)

        m4 = np.zeros((NP2, 1), np.float32)
        m4[np.asarray(jax.random.permutation(r4, N)[:NUM_MASK])] = 1.0
        m5 = np.zeros((NP2, 1), np.float32)
        m5[np.asarray(jax.random.permutation(r5, N)[:NUM_MASK])] = 1.0
    return (np.asarray(mask_nodes), np.asarray(keep_nodes), remap, m4, m5)


(_MASK_NODES, _KEEP_NODES, _REMAP, _M4, _M5) = _mask_constants_np()


def _sigmoid(x):
    return 1.0 / (1.0 + jnp.exp(-x))


def _bc_i32(x):
    return jax.lax.bitcast_convert_type(x, jnp.int32)


def _elu(x):
    return jnp.where(x > 0, x, jnp.exp(jnp.minimum(x, 0.0)) - 1.0)


def _k1_body(x_ref, w1l_ref, w1r_ref, w1le_ref, w1re_ref, as_ref, ad_ref,
             aes_ref, aed_ref, enc_ref,
             t_ref, b1_ref, c1_ref, be_ref, ce_ref, encr_ref):
    x = x_ref[...]
    f1 = jnp.dot(x, w1l_ref[...], preferred_element_type=jnp.float32)
    fd = jnp.dot(x, w1r_ref[...], preferred_element_type=jnp.float32)
    xe = jnp.dot(x, w1le_ref[...], preferred_element_type=jnp.float32)
    xde = jnp.dot(x, w1re_ref[...], preferred_element_type=jnp.float32)
    t_ref[...] = jnp.concatenate([f1, xe], axis=1)
    b1_ref[...] = jnp.sum(f1 * as_ref[...], axis=1, keepdims=True)
    c1_ref[...] = jnp.sum(fd * ad_ref[...], axis=1, keepdims=True)
    be_ref[...] = jnp.sum(xe * aes_ref[...], axis=1, keepdims=True)
    ce_ref[...] = jnp.sum(xde * aed_ref[...], axis=1, keepdims=True)
    enc = enc_ref[...]
    e1 = jnp.dot(enc, w1l_ref[...], preferred_element_type=jnp.float32)
    ed = jnp.dot(enc, w1r_ref[...], preferred_element_type=jnp.float32)
    encr_ref[...] = jnp.concatenate([e1, ed], axis=1)


def _run_k1(features, W1l, W1r, W1le, W1re, a_s, a_d, ae_s, ae_d, enc):
    grid = 10
    blk = N // grid
    full = lambda shp: pl.BlockSpec(shp, lambda i: (0, 0))
    return pl.pallas_call(
        _k1_body,
        grid=(grid,),
        in_specs=[
            pl.BlockSpec((blk, IN_DIM), lambda i: (i, 0)),
            full((IN_DIM, HID)), full((IN_DIM, HID)),
            full((IN_DIM, HID)), full((IN_DIM, HID)),
            full((1, HID)), full((1, HID)), full((1, HID)), full((1, HID)),
            full((8, IN_DIM)),
        ],
        out_specs=[
            pl.BlockSpec((blk, D2), lambda i: (i, 0)),
            pl.BlockSpec((blk, 1), lambda i: (i, 0)),
            pl.BlockSpec((blk, 1), lambda i: (i, 0)),
            pl.BlockSpec((blk, 1), lambda i: (i, 0)),
            pl.BlockSpec((blk, 1), lambda i: (i, 0)),
            full((8, D2)),
        ],
        out_shape=[
            jax.ShapeDtypeStruct((N, D2), jnp.float32),
            jax.ShapeDtypeStruct((N, 1), jnp.float32),
            jax.ShapeDtypeStruct((N, 1), jnp.float32),
            jax.ShapeDtypeStruct((N, 1), jnp.float32),
            jax.ShapeDtypeStruct((N, 1), jnp.float32),
            jax.ShapeDtypeStruct((8, D2), jnp.float32),
        ],
    )(features, W1l, W1r, W1le, W1re, a_s, a_d, ae_s, ae_d, enc)


def _sc_mesh():
    return plsc.VectorSubcoreMesh(core_axis_name="c", subcore_axis_name="s",
                                  num_cores=NC, num_subcores=NS)


def _k2_body(f1e, b1f, c1f, betab, cetab, remtab, srcdst,
             tmain_out, asrc_out, adst_out, a1_out, ae_out,
             rem_t, b1_t, c1_t, be_t, ce_t, idxb, bufa, bufb, asb, adb,
             eidx, a1b, aeb, sem1):
    cid = lax.axis_index("c")
    tid = lax.axis_index("s")
    wid = tid * NC + cid
    r0 = wid * RPW

    pltpu.sync_copy(remtab, rem_t)
    pltpu.sync_copy(b1f, b1_t)
    pltpu.sync_copy(c1f, c1_t)
    pltpu.sync_copy(betab, be_t)
    pltpu.sync_copy(cetab, ce_t)

    for k in range(RPW // BCH):
        rbase = r0 + k * BCH
        pltpu.sync_copy(remtab.at[pl.ds(rbase, BCH)], idxb.at[0])
        pltpu.async_copy(f1e.at[idxb.at[0]], bufa, sem1).wait()
        pltpu.sync_copy(f1e.at[pl.ds(rbase, BCH)], bufb)
        for r in range(BCH):
            for j in range(HID // 16):
                sl = pl.ds(j * 16, 16)
                bufb[r, sl] = bufa[r, sl]
        pltpu.sync_copy(bufb, tmain_out.at[pl.ds(rbase, BCH)])

    for g in range(RPW // 16):
        sl = pl.ds(g * 16, 16)
        rv = rem_t[pl.ds(r0 + g * 16, 16)]
        asb[sl] = plsc.load_gather(b1_t, [rv])
        adb[sl] = plsc.load_gather(c1_t, [rv])
    pltpu.sync_copy(asb, asrc_out.at[pl.ds(r0, RPW)])
    pltpu.sync_copy(adb, adst_out.at[pl.ds(r0, RPW)])

    ebase = wid * EPW

    def asup(si, _):
        base = ebase + si * SUPE
        pltpu.sync_copy(srcdst.at[wid * NSUP + si], eidx)

        def grp(g, _):
            sl = pl.ds(g * 16, 16)
            sv = eidx[pl.ds(g * 16, 16)]
            dv = eidx[pl.ds(SUPE + g * 16, 16)]
            svx = plsc.load_gather(rem_t, [sv])
            dvx = plsc.load_gather(rem_t, [dv])
            a1b[sl] = _sigmoid(plsc.load_gather(b1_t, [svx]) +
                               plsc.load_gather(c1_t, [dvx]))
            aeb[sl] = _sigmoid(plsc.load_gather(be_t, [sv]) +
                               plsc.load_gather(ce_t, [dv]))
            return 0

        lax.fori_loop(0, SUPE // 16, grp, 0)
        pltpu.sync_copy(a1b, a1_out.at[pl.ds(base, SUPE)])
        pltpu.sync_copy(aeb, ae_out.at[pl.ds(base, SUPE)])
        return 0

    lax.fori_loop(0, NSUP, asup, 0)


def _run_k2(f1e, b1f, c1f, betab, cetab, remtab, srcdst):
    return pl.kernel(
        _k2_body,
        out_type=[
            jax.ShapeDtypeStruct((NP2, D2), jnp.float32),
            jax.ShapeDtypeStruct((NP2,), jnp.float32),
            jax.ShapeDtypeStruct((NP2,), jnp.float32),
            jax.ShapeDtypeStruct((EP,), jnp.float32),
            jax.ShapeDtypeStruct((EP,), jnp.float32),
        ],
        mesh=_sc_mesh(),
        scratch_types=[
            pltpu.VMEM((NP2,), jnp.int32),
            pltpu.VMEM((NP2,), jnp.float32),
            pltpu.VMEM((NP2,), jnp.float32),
            pltpu.VMEM((NP2,), jnp.float32),
            pltpu.VMEM((NP2,), jnp.float32),
            pltpu.VMEM((1, BCH), jnp.int32),
            pltpu.VMEM((BCH, D2), jnp.float32),
            pltpu.VMEM((BCH, D2), jnp.float32),
            pltpu.VMEM((RPW,), jnp.float32),
            pltpu.VMEM((RPW,), jnp.float32),
            pltpu.VMEM((2 * SUPE,), jnp.int32),
            pltpu.VMEM((SUPE,), jnp.float32),
            pltpu.VMEM((SUPE,), jnp.float32),
            pltpu.SemaphoreType.DMA,
        ],
        **_SC_PARAMS,
    )(f1e, b1f, c1f, betab, cetab, remtab, srcdst)


def _edge_body(two_alpha, tmain, packed, acc_out,
               blk, rb0, rb1, acc, g0, g1, s0, s1):
    cid = lax.axis_index("c")
    tid = lax.axis_index("s")
    wid = tid * NC + cid

    z = jnp.zeros((16,), jnp.float32)

    def zrow(r, _):
        for j in range(D2 // 16):
            rb0[r, pl.ds(j * 16, 16)] = z
        return 0

    lax.fori_loop(0, CH, zrow, 0)
    for k in range(RPT // CH):
        pltpu.sync_copy(rb0, acc.at[pl.ds(tid * RPT + k * CH, CH)])
    plsc.subcore_barrier()

    def mult(rb, k):
        def egroup(eg, _):
            for u in range(4):
                e = eg * 4 + u
                av = plsc.bitcast(plsc.load_gather(
                    blk, [jnp.full((16,), 8 + k, jnp.int32),
                          jnp.full((16,), e, jnp.int32)]), jnp.float32)
                if two_alpha:
                    ev = plsc.bitcast(plsc.load_gather(
                        blk, [jnp.full((16,), 12 + k, jnp.int32),
                              jnp.full((16,), e, jnp.int32)]), jnp.float32)
                else:
                    ev = av
                for j in range(HID // 16):
                    sl = pl.ds(j * 16, 16)
                    rb[e, sl] = rb[e, sl] * av
                for j in range(HID // 16, D2 // 16):
                    sl = pl.ds(j * 16, 16)
                    rb[e, sl] = rb[e, sl] * ev
            return 0

        lax.fori_loop(0, CH // 4, egroup, 0)

    def gath(k, rb, sem):
        return pltpu.async_copy(tmain.at[blk.at[k]], rb, sem)

    def scat(k, rb, sem):
        return pltpu.async_copy(rb, acc.at[blk.at[4 + k]], sem, add=True)

    def sup(si, _):
        pltpu.sync_copy(packed.at[wid * NSUP + si], blk)
        cg0 = gath(0, rb0, g0)
        cg1 = gath(1, rb1, g1)
        cg0.wait()
        mult(rb0, 0)
        cs0 = scat(0, rb0, s0)
        cg1.wait()
        mult(rb1, 1)
        cs1 = scat(1, rb1, s1)
        cs0.wait()
        cg2 = gath(2, rb0, g0)
        cs1.wait()
        cg3 = gath(3, rb1, g1)
        cg2.wait()
        mult(rb0, 2)
        cs2 = scat(2, rb0, s0)
        cg3.wait()
        mult(rb1, 3)
        cs3 = scat(3, rb1, s1)
        cs2.wait()
        cs3.wait()
        return 0

    lax.fori_loop(0, NSUP, sup, 0)

    plsc.subcore_barrier()
    r0 = tid * RPT
    pltpu.sync_copy(acc.at[pl.ds(r0, RPT)], acc_out.at[cid, pl.ds(r0, RPT)])


def _run_edge_pass(two_alpha, tmain, packed):
    body = functools.partial(_edge_body, two_alpha)
    return pl.kernel(
        body,
        out_type=[jax.ShapeDtypeStruct((NC, NP2, D2), jnp.float32)],
        mesh=_sc_mesh(),
        scratch_types=[
            pltpu.VMEM((16, CH), jnp.int32),
            pltpu.VMEM((CH, D2), jnp.float32),
            pltpu.VMEM((CH, D2), jnp.float32),
            pltpu.VMEM_SHARED((NP2, D2), jnp.float32),
            pltpu.SemaphoreType.DMA,
            pltpu.SemaphoreType.DMA,
            pltpu.SemaphoreType.DMA,
            pltpu.SemaphoreType.DMA,
        ],
        **_SC_PARAMS,
    )(tmain, packed)


def _k4_body(a_ref, w2l_ref, w2le_ref, dec_ref, m4_ref, m5_ref,
             h2_ref, h2e_ref, t2_ref):
    a = a_ref[0] + a_ref[1]
    h1 = _elu(a[:, :HID])
    h1e = _elu(a[:, HID:])
    h2 = jnp.dot(h1, w2l_ref[...], preferred_element_type=jnp.float32)
    h2_ref[...] = h2
    h2e_ref[...] = jnp.dot(h1e, w2le_ref[...],
                           preferred_element_type=jnp.float32)
    m4 = m4_ref[...]
    m5 = m5_ref[...]
    dec = dec_ref[...]
    h2_1 = m4 * dec + (1.0 - m4) * h2
    h2_2 = m5 * dec + (1.0 - m5) * h2
    dn = (((1,), (1,)), ((), ()))
    x31 = lax.dot_general(h2_1, w2l_ref[...], dn,
                          preferred_element_type=jnp.float32)
    x32 = lax.dot_general(h2_2, w2l_ref[...], dn,
                          preferred_element_type=jnp.float32)
    t2_ref[...] = jnp.concatenate([x31, x32], axis=1)


def _run_k4(acc, W2l, W2le, dec, m4, m5):
    grid = 10
    blk = NP2 // grid
    full = lambda shp: pl.BlockSpec(shp, lambda i: (0, 0))
    return pl.pallas_call(
        _k4_body,
        grid=(grid,),
        in_specs=[
            pl.BlockSpec((NC, blk, D2), lambda i: (0, i, 0)),
            full((HID, OUT)), full((HID, OUT)), full((1, OUT)),
            pl.BlockSpec((blk, 1), lambda i: (i, 0)),
            pl.BlockSpec((blk, 1), lambda i: (i, 0)),
        ],
        out_specs=[
            pl.BlockSpec((blk, OUT), lambda i: (i, 0)),
            pl.BlockSpec((blk, OUT), lambda i: (i, 0)),
            pl.BlockSpec((blk, D2), lambda i: (i, 0)),
        ],
        out_shape=[
            jax.ShapeDtypeStruct((NP2, OUT), jnp.float32),
            jax.ShapeDtypeStruct((NP2, OUT), jnp.float32),
            jax.ShapeDtypeStruct((NP2, D2), jnp.float32),
        ],
    )(acc, W2l, W2le, dec, m4, m5)


def _k6_body(a_ref, w1l_ref, h41_ref, h42_ref):
    a = a_ref[0] + a_ref[1]
    o31 = _elu(a[:, :HID])
    o32 = _elu(a[:, HID:])
    dn = (((1,), (1,)), ((), ()))
    h41_ref[...] = lax.dot_general(o31, w1l_ref[...], dn,
                                   preferred_element_type=jnp.float32)
    h42_ref[...] = lax.dot_general(o32, w1l_ref[...], dn,
                                   preferred_element_type=jnp.float32)


def _run_k6(acc, W1l):
    grid = 10
    blk = NP2 // grid
    return pl.pallas_call(
        _k6_body,
        grid=(grid,),
        in_specs=[
            pl.BlockSpec((NC, blk, D2), lambda i: (0, i, 0)),
            pl.BlockSpec((IN_DIM, HID), lambda i: (0, 0)),
        ],
        out_specs=[
            pl.BlockSpec((blk, IN_DIM), lambda i: (i, 0)),
            pl.BlockSpec((blk, IN_DIM), lambda i: (i, 0)),
        ],
        out_shape=[
            jax.ShapeDtypeStruct((NP2, IN_DIM), jnp.float32),
            jax.ShapeDtypeStruct((NP2, IN_DIM), jnp.float32),
        ],
    )(acc, W1l)


def kernel(features, edge_index, enc_mask_token, dec_mask_token, W1l, W1r,
           att1_src, att1_dst, W2l, W2r, W1l_ema, W1r_ema, atte_src,
           atte_dst, W2l_ema, W2r_ema):
    mask_nodes = jnp.asarray(_MASK_NODES)
    keep_nodes = jnp.asarray(_KEEP_NODES)
    remtab = jnp.asarray(_REMAP)
    m4 = jnp.asarray(_M4)
    m5 = jnp.asarray(_M5)

    enc8 = jnp.broadcast_to(enc_mask_token, (8, IN_DIM))
    f1e, b1, c1, be, ce, encr = _run_k1(
        features, W1l, W1r, W1l_ema, W1r_ema,
        att1_src.reshape(1, HID), att1_dst.reshape(1, HID),
        atte_src.reshape(1, HID), atte_dst.reshape(1, HID), enc8)

    enc1 = encr[0, :HID]
    encd = encr[0, HID:]
    b1t = jnp.sum(enc1 * att1_src)
    c1t = jnp.sum(encd * att1_dst)

    padn = NP2 - N - 8
    encrow = jnp.concatenate(
        [jnp.broadcast_to(enc1, (8, HID)), jnp.zeros((8, HID), jnp.float32)],
        axis=1)
    f1e_pad = jnp.concatenate(
        [f1e, encrow, jnp.zeros((padn, D2), jnp.float32)], axis=0)

    def padtab(v, extra):
        return jnp.concatenate(
            [v[:, 0], jnp.full((8,), extra, jnp.float32),
             jnp.zeros((padn,), jnp.float32)], axis=0)

    b1f = padtab(b1, b1t)
    c1f = padtab(c1, c1t)
    betab = jnp.concatenate([be[:, 0], jnp.zeros((NP2 - N,), jnp.float32)])
    cetab = jnp.concatenate([ce[:, 0], jnp.zeros((NP2 - N,), jnp.float32)])

    epad = jnp.full((EP - E,), DUMP_ROW, jnp.int32)
    srcp = jnp.concatenate([edge_index[0], epad])
    dstp = jnp.concatenate([edge_index[1], epad])
    srcdst = jnp.concatenate(
        [srcp.reshape(NSUPT, 1, SUPE), dstp.reshape(NSUPT, 1, SUPE)],
        axis=1).reshape(NSUPT, 2 * SUPE)

    tmain, a_src, a_dst, alpha1, alphae = _run_k2(
        f1e_pad, b1f, c1f, betab, cetab, remtab, srcdst)

    packed = jnp.concatenate(
        [srcp.reshape(NSUPT, SB, CH),
         dstp.reshape(NSUPT, SB, CH),
         _bc_i32(alpha1).reshape(NSUPT, SB, CH),
         _bc_i32(alphae).reshape(NSUPT, SB, CH)], axis=1)

    [acca] = _run_edge_pass(True, tmain, packed)

    h2p, h2ep, t2 = _run_k4(acca, W2l, W2l_ema, dec_mask_token, m4, m5)

    [accb] = _run_edge_pass(False, t2, packed)

    h41p, h42p = _run_k6(accb, W1l)

    return (h2p[:N], h2ep[:N], h41p[:N], h42p[:N], mask_nodes, keep_nodes,
            (a_src[:N], a_dst[:N]))
